# Initial kernel scaffold; baseline (speedup 1.0000x reference)
import jax
import jax.numpy as jnp
from jax.experimental import pallas as pl


def kernel(x, edge_index, W1, b1, W2, b2, Wd, bd):
    raise NotImplementedError("write your pallas kernel here")


# speedup vs baseline: 9.5181x; 9.5181x over previous
import functools

import jax
import jax.numpy as jnp
from jax import lax
from jax.experimental import pallas as pl
from jax.experimental.pallas import tpu as pltpu
from jax.experimental.pallas import tpu_sc as plsc

N_NODES = 10000
N_EDGES = 320000
D = 128

NC = 2
NS = 16
NW = NC * NS

N_PAD = 10240
ROWS_PER_TILE = N_PAD // NS
CHUNK = 128
CHUNKS_PER_W = 80
E_PAD = NW * CHUNKS_PER_W * CHUNK
F32 = jnp.float32

_mesh = plsc.VectorSubcoreMesh(core_axis_name="c", subcore_axis_name="s")


@functools.partial(
    pl.kernel,
    out_type=jax.ShapeDtypeStruct((NW * N_PAD,), F32),
    mesh=_mesh,
    compiler_params=pltpu.CompilerParams(needs_layout_passes=False),
    scratch_types=[
        pltpu.VMEM((CHUNKS_PER_W, CHUNK), jnp.int32),
        pltpu.VMEM((N_PAD,), F32),
    ],
)
def _deg_kernel(dst_hbm, out_hbm, dst_v, deg_v):
    c = lax.axis_index("c")
    s = lax.axis_index("s")
    wid = c * NS + s
    pltpu.sync_copy(dst_hbm.at[pl.ds(wid * CHUNKS_PER_W, CHUNKS_PER_W)], dst_v)

    zeros16 = jnp.zeros((16,), F32)

    def zero_body(i, _):
        deg_v[pl.ds(i * 16, 16)] = zeros16
        return 0

    lax.fori_loop(0, N_PAD // 16, zero_body, 0)

    ones16 = jnp.ones((16,), F32)
    n_groups = CHUNKS_PER_W * (CHUNK // 16)

    def acc_body(i, _):
        j = i // (CHUNK // 16)
        k = i % (CHUNK // 16)
        idx = dst_v[j, pl.ds(k * 16, 16)]
        plsc.addupdate_scatter(deg_v, [idx], ones16)
        return 0

    lax.fori_loop(0, n_groups, acc_body, 0)
    pltpu.sync_copy(deg_v, out_hbm.at[pl.ds(wid * N_PAD, N_PAD)])


@functools.partial(
    pl.kernel,
    out_type=jax.ShapeDtypeStruct((NC, N_PAD, D), F32),
    mesh=_mesh,
    scratch_types=[
        pltpu.VMEM((CHUNKS_PER_W, CHUNK), jnp.int32),
        pltpu.VMEM((CHUNKS_PER_W, CHUNK), jnp.int32),
        pltpu.VMEM((CHUNK, D), F32),
        pltpu.VMEM_SHARED((N_PAD, D), F32),
        pltpu.SemaphoreType.DMA,
    ],
)
def _prop_kernel(src_hbm, dst_hbm, y_hbm, z_hbm, out_hbm,
                 src_v, dst_v, rows_v, acc, sem):
    c = lax.axis_index("c")
    s = lax.axis_index("s")
    wid = c * NS + s
    row0 = s * ROWS_PER_TILE

    @pl.when(c == 0)
    def _():
        pltpu.sync_copy(y_hbm.at[pl.ds(row0, ROWS_PER_TILE)],
                        acc.at[pl.ds(row0, ROWS_PER_TILE)])

    @pl.when(c == 1)
    def _():
        pltpu.sync_copy(z_hbm.at[pl.ds(row0, ROWS_PER_TILE)],
                        acc.at[pl.ds(row0, ROWS_PER_TILE)])

    pltpu.sync_copy(src_hbm.at[pl.ds(wid * CHUNKS_PER_W, CHUNKS_PER_W)], src_v)
    pltpu.sync_copy(dst_hbm.at[pl.ds(wid * CHUNKS_PER_W, CHUNKS_PER_W)], dst_v)
    plsc.subcore_barrier()

    def body(j, _):
        pltpu.async_copy(y_hbm.at[src_v.at[j]], rows_v, sem).wait()
        pltpu.sync_copy(rows_v, acc.at[dst_v.at[j]], add=True)
        return 0

    lax.fori_loop(0, CHUNKS_PER_W, body, 0)
    plsc.subcore_barrier()
    pltpu.sync_copy(acc.at[pl.ds(row0, ROWS_PER_TILE)],
                    out_hbm.at[c, pl.ds(row0, ROWS_PER_TILE)])


_BLK = 1280
_GRID = N_PAD // _BLK


def _tc1_body(degp_ref, x_ref, w1_ref, dinv_ref, y_ref):
    deg = jnp.sum(degp_ref[...], axis=0) + 1.0
    dinv = lax.rsqrt(deg)
    xl = lax.dot_general(x_ref[...], w1_ref[...], (((1,), (1,)), ((), ())),
                         preferred_element_type=F32)
    y_ref[...] = xl * dinv[:, None]
    dinv_ref[...] = dinv[:, None]


def _tc2_body(p_ref, dinv_ref, b1_ref, w2_ref, y2_ref):
    t = (p_ref[0] + p_ref[1]) * dinv_ref[...] + b1_ref[...]
    h = jnp.where(t >= 0, t, 0.01 * t)
    xl = lax.dot_general(h, w2_ref[...], (((1,), (1,)), ((), ())),
                         preferred_element_type=F32)
    y2_ref[...] = xl * dinv_ref[...]


def _tc3_body(q_ref, dinv_ref, b2_ref, wd_ref, bd_ref, out_ref):
    t = (q_ref[0] + q_ref[1]) * dinv_ref[...] + b2_ref[...]
    h = jnp.where(t >= 0, t, 0.01 * t)
    out_ref[...] = jnp.sum(h * wd_ref[...], axis=1, keepdims=True) + bd_ref[0]


def kernel(x, edge_index, W1, b1, W2, b2, Wd, bd):
    ei = edge_index.astype(jnp.int32)
    pad_e = jnp.full((E_PAD - N_EDGES,), N_NODES, jnp.int32)
    src2d = jnp.concatenate([ei[0], pad_e]).reshape(NW * CHUNKS_PER_W, CHUNK)
    dst2d = jnp.concatenate([ei[1], pad_e]).reshape(NW * CHUNKS_PER_W, CHUNK)
    x_pad = jnp.zeros((N_PAD, D), F32).at[:N_NODES].set(x)
    zeros_nd = jnp.zeros((N_PAD, D), F32)

    degp = _deg_kernel(dst2d).reshape(NW, N_PAD)

    dinv, y1 = pl.pallas_call(
        _tc1_body,
        grid=(_GRID,),
        in_specs=[
            pl.BlockSpec((NW, _BLK), lambda i: (0, i)),
            pl.BlockSpec((_BLK, D), lambda i: (i, 0)),
            pl.BlockSpec((D, D), lambda i: (0, 0)),
        ],
        out_specs=[
            pl.BlockSpec((_BLK, 1), lambda i: (i, 0)),
            pl.BlockSpec((_BLK, D), lambda i: (i, 0)),
        ],
        out_shape=[
            jax.ShapeDtypeStruct((N_PAD, 1), F32),
            jax.ShapeDtypeStruct((N_PAD, D), F32),
        ],
    )(degp, x_pad, W1)

    p = _prop_kernel(src2d, dst2d, y1, zeros_nd)

    y2 = pl.pallas_call(
        _tc2_body,
        grid=(_GRID,),
        in_specs=[
            pl.BlockSpec((NC, _BLK, D), lambda i: (0, i, 0)),
            pl.BlockSpec((_BLK, 1), lambda i: (i, 0)),
            pl.BlockSpec((1, D), lambda i: (0, 0)),
            pl.BlockSpec((D, D), lambda i: (0, 0)),
        ],
        out_specs=pl.BlockSpec((_BLK, D), lambda i: (i, 0)),
        out_shape=jax.ShapeDtypeStruct((N_PAD, D), F32),
    )(p, dinv, b1.reshape(1, D), W2)

    q = _prop_kernel(src2d, dst2d, y2, zeros_nd)

    out = pl.pallas_call(
        _tc3_body,
        grid=(_GRID,),
        in_specs=[
            pl.BlockSpec((NC, _BLK, D), lambda i: (0, i, 0)),
            pl.BlockSpec((_BLK, 1), lambda i: (i, 0)),
            pl.BlockSpec((1, D), lambda i: (0, 0)),
            pl.BlockSpec((1, D), lambda i: (0, 0)),
            pl.BlockSpec(memory_space=pltpu.MemorySpace.SMEM),
        ],
        out_specs=pl.BlockSpec((_BLK, 1), lambda i: (i, 0)),
        out_shape=jax.ShapeDtypeStruct((N_PAD, 1), F32),
    )(q, dinv, b2.reshape(1, D), Wd, bd)

    return out[:N_NODES]


# speedup vs baseline: 32.6612x; 3.4315x over previous
import functools

import jax
import jax.numpy as jnp
from jax import lax
from jax.experimental import pallas as pl
from jax.experimental.pallas import tpu as pltpu
from jax.experimental.pallas import tpu_sc as plsc

N_NODES = 10000
N_EDGES = 320000
D = 128

NC = 2
NS = 16
NW = NC * NS

N_PAD = 10240
ROWS_PER_TILE = N_PAD // NS
CHUNK = 128
CHUNKS_PER_W = 80
E_PAD = NW * CHUNKS_PER_W * CHUNK
G = 8
NPAIR = CHUNKS_PER_W // (2 * G)
F32 = jnp.float32

_mesh = plsc.VectorSubcoreMesh(core_axis_name="c", subcore_axis_name="s")


@functools.partial(
    pl.kernel,
    out_type=jax.ShapeDtypeStruct((NW * N_PAD,), F32),
    mesh=_mesh,
    compiler_params=pltpu.CompilerParams(needs_layout_passes=False),
    scratch_types=[
        pltpu.VMEM((CHUNKS_PER_W, 2, CHUNK), jnp.int32),
        pltpu.VMEM((N_PAD,), F32),
    ],
)
def _deg_kernel(idx_hbm, out_hbm, idx_v, deg_v):
    c = lax.axis_index("c")
    s = lax.axis_index("s")
    wid = c * NS + s
    pltpu.sync_copy(idx_hbm.at[pl.ds(wid * CHUNKS_PER_W, CHUNKS_PER_W)], idx_v)

    zeros16 = jnp.zeros((16,), F32)

    def zero_body(i, _):
        deg_v[pl.ds(i * 16, 16)] = zeros16
        return 0

    lax.fori_loop(0, N_PAD // 16, zero_body, 0)

    ones16 = jnp.ones((16,), F32)
    n_groups = CHUNKS_PER_W * (CHUNK // 16)

    def acc_body(i, _):
        j = i // (CHUNK // 16)
        k = i % (CHUNK // 16)
        idx = idx_v[j, 1, pl.ds(k * 16, 16)]
        plsc.addupdate_scatter(deg_v, [idx], ones16)
        return 0

    lax.fori_loop(0, n_groups, acc_body, 0)
    pltpu.sync_copy(deg_v, out_hbm.at[pl.ds(wid * N_PAD, N_PAD)])


@functools.partial(
    pl.kernel,
    out_type=jax.ShapeDtypeStruct((NC, N_PAD, D), F32),
    mesh=_mesh,
    scratch_types=[
        pltpu.VMEM((G, 2, CHUNK), jnp.int32),
        pltpu.VMEM((G, 2, CHUNK), jnp.int32),
        pltpu.VMEM((CHUNK, D), F32),
        pltpu.VMEM((CHUNK, D), F32),
        pltpu.VMEM_SHARED((N_PAD, D), F32),
        pltpu.SemaphoreType.DMA,
        pltpu.SemaphoreType.DMA,
        pltpu.SemaphoreType.DMA,
        pltpu.SemaphoreType.DMA,
    ],
)
def _prop_kernel(idx_hbm, y_hbm, z_hbm, out_hbm,
                 idxa_v, idxb_v, rows0_v, rows1_v, acc,
                 sema, semb, sem0, sem1):
    c = lax.axis_index("c")
    s = lax.axis_index("s")
    wid = c * NS + s
    base = wid * CHUNKS_PER_W
    row0 = s * ROWS_PER_TILE

    @pl.when(c == 0)
    def _():
        pltpu.sync_copy(y_hbm.at[pl.ds(row0, ROWS_PER_TILE)],
                        acc.at[pl.ds(row0, ROWS_PER_TILE)])

    @pl.when(c == 1)
    def _():
        pltpu.sync_copy(z_hbm.at[pl.ds(row0, ROWS_PER_TILE)],
                        acc.at[pl.ds(row0, ROWS_PER_TILE)])

    pltpu.async_copy(idx_hbm.at[pl.ds(base, G)], idxa_v, sema)
    plsc.subcore_barrier()

    rows = [rows0_v, rows1_v]
    sems = [sem0, sem1]

    def body(i, _):
        g0 = 2 * i * G
        pltpu.make_async_copy(idx_hbm.at[pl.ds(base, G)], idxa_v, sema).wait()
        pltpu.async_copy(idx_hbm.at[pl.ds(base + g0 + G, G)], idxb_v, semb)
        pltpu.async_copy(y_hbm.at[idxa_v.at[0, 0]], rows0_v, sem0)
        for r in range(G):
            if r < G - 1:
                pltpu.async_copy(y_hbm.at[idxa_v.at[r + 1, 0]],
                                 rows[(r + 1) % 2], sems[(r + 1) % 2])
            else:
                pltpu.make_async_copy(idx_hbm.at[pl.ds(base, G)],
                                      idxb_v, semb).wait()
                pltpu.async_copy(y_hbm.at[idxb_v.at[0, 0]],
                                 rows[G % 2], sems[G % 2])
            pltpu.make_async_copy(y_hbm.at[pl.ds(0, CHUNK)], rows[r % 2], sems[r % 2]).wait()
            pltpu.sync_copy(rows[r % 2], acc.at[idxa_v.at[r, 1]], add=True)

        @pl.when(i + 1 < NPAIR)
        def _():
            pltpu.async_copy(idx_hbm.at[pl.ds(base + g0 + 2 * G, G)],
                             idxa_v, sema)

        for r in range(G, 2 * G):
            if r < 2 * G - 1:
                pltpu.async_copy(y_hbm.at[idxb_v.at[r - G + 1, 0]],
                                 rows[(r + 1) % 2], sems[(r + 1) % 2])
            pltpu.make_async_copy(y_hbm.at[pl.ds(0, CHUNK)], rows[r % 2], sems[r % 2]).wait()
            pltpu.sync_copy(rows[r % 2], acc.at[idxb_v.at[r - G, 1]], add=True)
        return 0

    lax.fori_loop(0, NPAIR, body, 0)
    plsc.subcore_barrier()
    pltpu.sync_copy(acc.at[pl.ds(row0, ROWS_PER_TILE)],
                    out_hbm.at[c, pl.ds(row0, ROWS_PER_TILE)])


_BLK = 1280
_GRID = N_PAD // _BLK


def _tc1_body(degp_ref, x_ref, w1_ref, dinv_ref, y_ref):
    deg = jnp.sum(degp_ref[...], axis=0) + 1.0
    dinv = lax.rsqrt(deg)
    xl = lax.dot_general(x_ref[...], w1_ref[...], (((1,), (1,)), ((), ())),
                         preferred_element_type=F32)
    y_ref[...] = xl * dinv[:, None]
    dinv_ref[...] = dinv[:, None]


def _tc2_body(p_ref, dinv_ref, b1_ref, w2_ref, y2_ref):
    t = (p_ref[0] + p_ref[1]) * dinv_ref[...] + b1_ref[...]
    h = jnp.where(t >= 0, t, 0.01 * t)
    xl = lax.dot_general(h, w2_ref[...], (((1,), (1,)), ((), ())),
                         preferred_element_type=F32)
    y2_ref[...] = xl * dinv_ref[...]


def _tc3_body(q_ref, dinv_ref, b2_ref, wd_ref, bd_ref, out_ref):
    t = (q_ref[0] + q_ref[1]) * dinv_ref[...] + b2_ref[...]
    h = jnp.where(t >= 0, t, 0.01 * t)
    out_ref[...] = jnp.sum(h * wd_ref[...], axis=1, keepdims=True) + bd_ref[0]


def kernel(x, edge_index, W1, b1, W2, b2, Wd, bd):
    ei = edge_index.astype(jnp.int32)
    pad_e = N_NODES + (jnp.arange(E_PAD - N_EDGES, dtype=jnp.int32)
                       % (N_PAD - N_NODES))
    src2d = jnp.concatenate([ei[0], pad_e]).reshape(NW * CHUNKS_PER_W, CHUNK)
    dst2d = jnp.concatenate([ei[1], pad_e]).reshape(NW * CHUNKS_PER_W, CHUNK)
    idx2d = jnp.stack([src2d, dst2d], axis=1)
    x_pad = jnp.zeros((N_PAD, D), F32).at[:N_NODES].set(x)
    zeros_nd = jnp.zeros((N_PAD, D), F32)

    degp = _deg_kernel(idx2d).reshape(NW, N_PAD)

    dinv, y1 = pl.pallas_call(
        _tc1_body,
        grid=(_GRID,),
        in_specs=[
            pl.BlockSpec((NW, _BLK), lambda i: (0, i)),
            pl.BlockSpec((_BLK, D), lambda i: (i, 0)),
            pl.BlockSpec((D, D), lambda i: (0, 0)),
        ],
        out_specs=[
            pl.BlockSpec((_BLK, 1), lambda i: (i, 0)),
            pl.BlockSpec((_BLK, D), lambda i: (i, 0)),
        ],
        out_shape=[
            jax.ShapeDtypeStruct((N_PAD, 1), F32),
            jax.ShapeDtypeStruct((N_PAD, D), F32),
        ],
    )(degp, x_pad, W1)

    p = _prop_kernel(idx2d, y1, zeros_nd)

    y2 = pl.pallas_call(
        _tc2_body,
        grid=(_GRID,),
        in_specs=[
            pl.BlockSpec((NC, _BLK, D), lambda i: (0, i, 0)),
            pl.BlockSpec((_BLK, 1), lambda i: (i, 0)),
            pl.BlockSpec((1, D), lambda i: (0, 0)),
            pl.BlockSpec((D, D), lambda i: (0, 0)),
        ],
        out_specs=pl.BlockSpec((_BLK, D), lambda i: (i, 0)),
        out_shape=jax.ShapeDtypeStruct((N_PAD, D), F32),
    )(p, dinv, b1.reshape(1, D), W2)

    q = _prop_kernel(idx2d, y2, zeros_nd)

    out = pl.pallas_call(
        _tc3_body,
        grid=(_GRID,),
        in_specs=[
            pl.BlockSpec((NC, _BLK, D), lambda i: (0, i, 0)),
            pl.BlockSpec((_BLK, 1), lambda i: (i, 0)),
            pl.BlockSpec((1, D), lambda i: (0, 0)),
            pl.BlockSpec((1, D), lambda i: (0, 0)),
            pl.BlockSpec(memory_space=pltpu.MemorySpace.SMEM),
        ],
        out_specs=pl.BlockSpec((_BLK, 1), lambda i: (i, 0)),
        out_shape=jax.ShapeDtypeStruct((N_PAD, 1), F32),
    )(q, dinv, b2.reshape(1, D), Wd, bd)

    return out[:N_NODES]


# speedup vs baseline: 33.6003x; 1.0288x over previous
import functools

import jax
import jax.numpy as jnp
from jax import lax
from jax.experimental import pallas as pl
from jax.experimental.pallas import tpu as pltpu
from jax.experimental.pallas import tpu_sc as plsc

N_NODES = 10000
N_EDGES = 320000
D = 128

NC = 2
NS = 16
NW = NC * NS

N_PAD = 10240
ROWS_PER_TILE = N_PAD // NS
CHUNK = 128
CHUNKS_PER_W = 80
E_PAD = NW * CHUNKS_PER_W * CHUNK
G = 8
NPAIR = CHUNKS_PER_W // (2 * G)
F32 = jnp.float32

_mesh = plsc.VectorSubcoreMesh(core_axis_name="c", subcore_axis_name="s")


@functools.partial(
    pl.kernel,
    out_type=jax.ShapeDtypeStruct((NW * N_PAD,), F32),
    mesh=_mesh,
    compiler_params=pltpu.CompilerParams(needs_layout_passes=False),
    scratch_types=[
        pltpu.VMEM((CHUNKS_PER_W, CHUNK), jnp.int32),
        pltpu.VMEM((N_PAD,), F32),
    ],
)
def _deg_kernel(dst_hbm, out_hbm, dst_v, deg_v):
    c = lax.axis_index("c")
    s = lax.axis_index("s")
    wid = c * NS + s
    pltpu.sync_copy(dst_hbm.at[pl.ds(wid * CHUNKS_PER_W, CHUNKS_PER_W)], dst_v)

    zeros16 = jnp.zeros((16,), F32)

    def zero_body(i, _):
        deg_v[pl.ds(i * 16, 16)] = zeros16
        return 0

    lax.fori_loop(0, N_PAD // 16, zero_body, 0)

    ones16 = jnp.ones((16,), F32)
    n_groups = CHUNKS_PER_W * (CHUNK // 16)

    def acc_body(i, _):
        j = i // (CHUNK // 16)
        k = i % (CHUNK // 16)
        idx = dst_v[j, pl.ds(k * 16, 16)]
        plsc.addupdate_scatter(deg_v, [idx], ones16)
        return 0

    lax.fori_loop(0, n_groups, acc_body, 0)
    pltpu.sync_copy(deg_v, out_hbm.at[pl.ds(wid * N_PAD, N_PAD)])


@functools.partial(
    pl.kernel,
    out_type=jax.ShapeDtypeStruct((NC, N_PAD, D), F32),
    mesh=_mesh,
    scratch_types=[
        pltpu.VMEM((G, CHUNK), jnp.int32),
        pltpu.VMEM((G, CHUNK), jnp.int32),
        pltpu.VMEM((G, CHUNK), jnp.int32),
        pltpu.VMEM((G, CHUNK), jnp.int32),
        pltpu.VMEM((CHUNK, D), F32),
        pltpu.VMEM((CHUNK, D), F32),
        pltpu.VMEM_SHARED((N_PAD, D), F32),
        pltpu.SemaphoreType.DMA,
        pltpu.SemaphoreType.DMA,
        pltpu.SemaphoreType.DMA,
        pltpu.SemaphoreType.DMA,
    ],
)
def _prop_kernel(src_hbm, dst_hbm, y_hbm, out_hbm,
                 srca_v, dsta_v, srcb_v, dstb_v, rows0_v, rows1_v, acc,
                 sema, semb, sem0, sem1):
    c = lax.axis_index("c")
    s = lax.axis_index("s")
    wid = c * NS + s
    base = wid * CHUNKS_PER_W
    row0 = s * ROWS_PER_TILE

    pltpu.async_copy(src_hbm.at[pl.ds(base, G)], srca_v, sema)
    pltpu.async_copy(dst_hbm.at[pl.ds(base, G)], dsta_v, sema)

    @pl.when(c == 0)
    def _():
        pltpu.sync_copy(y_hbm.at[pl.ds(row0, ROWS_PER_TILE)],
                        acc.at[pl.ds(row0, ROWS_PER_TILE)])

    @pl.when(c == 1)
    def _():
        zeros16 = jnp.zeros((16,), F32)

        def zb(i, _):
            rows0_v[i // (D // 16), pl.ds((i % (D // 16)) * 16, 16)] = zeros16
            return 0

        lax.fori_loop(0, CHUNK * (D // 16), zb, 0)
        for t in range(ROWS_PER_TILE // CHUNK):
            pltpu.sync_copy(rows0_v, acc.at[pl.ds(row0 + t * CHUNK, CHUNK)])

    plsc.subcore_barrier()

    rows = [rows0_v, rows1_v]
    sems = [sem0, sem1]
    srcs = [srca_v, srcb_v]
    dsts = [dsta_v, dstb_v]

    def _wait(sem, dst):
        pltpu.make_async_copy(y_hbm.at[pl.ds(0, G)], dst, sem).wait()

    def body(i, _):
        g0 = 2 * i * G
        _wait(sema, srca_v)
        _wait(sema, dsta_v)
        pltpu.async_copy(src_hbm.at[pl.ds(base + g0 + G, G)], srcb_v, semb)
        pltpu.async_copy(dst_hbm.at[pl.ds(base + g0 + G, G)], dstb_v, semb)
        pltpu.async_copy(y_hbm.at[srca_v.at[0]], rows0_v, sem0)
        for half in range(2):
            src_v, dst_v = srcs[half], dsts[half]
            for r in range(G):
                rr = half * G + r
                if r < G - 1:
                    pltpu.async_copy(y_hbm.at[src_v.at[r + 1]],
                                     rows[(rr + 1) % 2], sems[(rr + 1) % 2])
                elif half == 0:
                    _wait(semb, srcb_v)
                    _wait(semb, dstb_v)
                    pltpu.async_copy(y_hbm.at[srcb_v.at[0]],
                                     rows[(rr + 1) % 2], sems[(rr + 1) % 2])
                pltpu.make_async_copy(y_hbm.at[pl.ds(0, CHUNK)],
                                      rows[rr % 2], sems[rr % 2]).wait()
                pltpu.sync_copy(rows[rr % 2], acc.at[dst_v.at[r]], add=True)
            if half == 0:
                @pl.when(i + 1 < NPAIR)
                def _():
                    pltpu.async_copy(src_hbm.at[pl.ds(base + g0 + 2 * G, G)],
                                     srca_v, sema)
                    pltpu.async_copy(dst_hbm.at[pl.ds(base + g0 + 2 * G, G)],
                                     dsta_v, sema)
        return 0

    lax.fori_loop(0, NPAIR, body, 0)
    plsc.subcore_barrier()
    pltpu.sync_copy(acc.at[pl.ds(row0, ROWS_PER_TILE)],
                    out_hbm.at[c, pl.ds(row0, ROWS_PER_TILE)])


_BLK = 1280
_GRID = N_PAD // _BLK


def _tc1_body(degp_ref, x_ref, w1_ref, dinv_ref, y_ref):
    deg = jnp.sum(degp_ref[...], axis=0) + 1.0
    dinv = lax.rsqrt(deg)
    xl = lax.dot_general(x_ref[...], w1_ref[...], (((1,), (1,)), ((), ())),
                         preferred_element_type=F32)
    y_ref[...] = xl * dinv[:, None]
    dinv_ref[...] = dinv[:, None]


def _tc2_body(p_ref, dinv_ref, b1_ref, w2_ref, y2_ref):
    t = (p_ref[0] + p_ref[1]) * dinv_ref[...] + b1_ref[...]
    h = jnp.where(t >= 0, t, 0.01 * t)
    xl = lax.dot_general(h, w2_ref[...], (((1,), (1,)), ((), ())),
                         preferred_element_type=F32)
    y2_ref[...] = xl * dinv_ref[...]


def _tc3_body(q_ref, dinv_ref, b2_ref, wd_ref, bd_ref, out_ref):
    t = (q_ref[0] + q_ref[1]) * dinv_ref[...] + b2_ref[...]
    h = jnp.where(t >= 0, t, 0.01 * t)
    out_ref[...] = jnp.sum(h * wd_ref[...], axis=1, keepdims=True) + bd_ref[0]


def kernel(x, edge_index, W1, b1, W2, b2, Wd, bd):
    ei = edge_index.astype(jnp.int32)
    pad_e = N_NODES + (jnp.arange(E_PAD - N_EDGES, dtype=jnp.int32)
                       % (N_PAD - N_NODES))
    src2d = jnp.concatenate([ei[0], pad_e]).reshape(NW * CHUNKS_PER_W, CHUNK)
    dst2d = jnp.concatenate([ei[1], pad_e]).reshape(NW * CHUNKS_PER_W, CHUNK)

    degp = _deg_kernel(dst2d).reshape(NW, N_PAD)

    dinv, y1 = pl.pallas_call(
        _tc1_body,
        grid=(_GRID,),
        in_specs=[
            pl.BlockSpec((NW, _BLK), lambda i: (0, i)),
            pl.BlockSpec((_BLK, D), lambda i: (i, 0)),
            pl.BlockSpec((D, D), lambda i: (0, 0)),
        ],
        out_specs=[
            pl.BlockSpec((_BLK, 1), lambda i: (i, 0)),
            pl.BlockSpec((_BLK, D), lambda i: (i, 0)),
        ],
        out_shape=[
            jax.ShapeDtypeStruct((N_PAD, 1), F32),
            jax.ShapeDtypeStruct((N_PAD, D), F32),
        ],
    )(degp, x, W1)

    p = _prop_kernel(src2d, dst2d, y1)

    y2 = pl.pallas_call(
        _tc2_body,
        grid=(_GRID,),
        in_specs=[
            pl.BlockSpec((NC, _BLK, D), lambda i: (0, i, 0)),
            pl.BlockSpec((_BLK, 1), lambda i: (i, 0)),
            pl.BlockSpec((1, D), lambda i: (0, 0)),
            pl.BlockSpec((D, D), lambda i: (0, 0)),
        ],
        out_specs=pl.BlockSpec((_BLK, D), lambda i: (i, 0)),
        out_shape=jax.ShapeDtypeStruct((N_PAD, D), F32),
    )(p, dinv, b1.reshape(1, D), W2)

    q = _prop_kernel(src2d, dst2d, y2)

    out = pl.pallas_call(
        _tc3_body,
        grid=(_GRID,),
        in_specs=[
            pl.BlockSpec((NC, _BLK, D), lambda i: (0, i, 0)),
            pl.BlockSpec((_BLK, 1), lambda i: (i, 0)),
            pl.BlockSpec((1, D), lambda i: (0, 0)),
            pl.BlockSpec((1, D), lambda i: (0, 0)),
            pl.BlockSpec(memory_space=pltpu.MemorySpace.SMEM),
        ],
        out_specs=pl.BlockSpec((_BLK, 1), lambda i: (i, 0)),
        out_shape=jax.ShapeDtypeStruct((N_NODES, 1), F32),
    )(q, dinv, b2.reshape(1, D), Wd, bd)

    return out
